# Initial kernel scaffold; baseline (speedup 1.0000x reference)
import jax
import jax.numpy as jnp
from jax.experimental import pallas as pl


def kernel(x):
    raise NotImplementedError("write your pallas kernel here")


# speedup vs baseline: 13.6399x; 13.6399x over previous
import jax
import jax.numpy as jnp
from jax import lax
from jax.experimental import pallas as pl
from jax.experimental.pallas import tpu as pltpu
from jax.experimental.pallas import tpu_sc as plsc

_B, _S, _D = 4, 8192, 2048
_K = 64
_L = 16
_PCAP = 64
_CHECK = 32
_FB = 128
_SB = _FB // _L
_CH = 256
_NCH = _S // _CH
_NWIN = _CH // _CHECK
_NC, _NS = 2, 16
_NW = _NC * _NS
_DBLK = _D // _FB
_GROUPS = _B * _DBLK
_GPW = _GROUPS // _NW
_NEG = float("-inf")


def _ce(vals, i, j):
    a, b = vals[i], vals[j]
    vals[i] = jnp.maximum(a, b)
    vals[j] = jnp.minimum(a, b)


def _sort_desc(vals):
    n = len(vals)
    k = 2
    while k <= n:
        j = k // 2
        while j >= 1:
            for i in range(n):
                l = i ^ j
                if l > i:
                    if (i & k) == 0:
                        _ce(vals, i, l)
                    else:
                        _ce(vals, l, i)
            j //= 2
        k *= 2


def _merge_topk(r, p):
    n = len(r)
    m = [jnp.maximum(r[i], p[n - 1 - i]) for i in range(n)]
    j = n // 2
    while j >= 1:
        for i in range(n):
            l = i ^ j
            if l > i:
                _ce(m, i, l)
        j //= 2
    return m


def _kmax_body(x_hbm, out_hbm, buf, pend, top, obuf, cbuf):
    cid = lax.axis_index("c")
    sid = lax.axis_index("s")
    wid = sid * _NC + cid
    lanes = lax.iota(jnp.int32, _L)
    neg = jnp.full((_L,), _NEG, jnp.float32)

    def group_body(g, carry):
        gid = wid * _GPW + g
        b = gid // _DBLK
        d0 = (gid % _DBLK) * _FB

        def init_body(i, carry):
            top[i] = neg
            pend[pl.ds(i * _L, _L)] = neg
            return carry

        lax.fori_loop(0, _SB * _K, init_body, 0)
        for sb in range(_SB):
            cbuf[sb] = jnp.zeros((_L,), jnp.int32)

        def chunk_body(ci, carry):
            pltpu.sync_copy(
                x_hbm.at[b, pl.ds(ci * _CH, _CH), pl.ds(d0, _FB)], buf
            )

            def sb_body(sb, carry):
                base = sb * _K
                pbase = sb * (_PCAP * _L)
                lane_off = lanes + pbase

                def flush():
                    p = [pend[pl.ds(pbase + i * _L, _L)] for i in range(_PCAP)]
                    _sort_desc(p)
                    r = [top[base + i] for i in range(_K)]
                    new = _merge_topk(r, p)
                    for i in range(_K):
                        top[base + i] = new[i]
                    for i in range(_PCAP):
                        pend[pl.ds(pbase + i * _L, _L)] = neg

                def win_body(w, ct):
                    c, t = ct
                    for ri in range(_CHECK):
                        v = buf[w * _CHECK + ri, pl.ds(sb * _L, _L)]
                        m = v > t
                        plsc.store_scatter(
                            pend, [lane_off + c * _L], v, mask=m
                        )
                        c = c + m.astype(jnp.int32)
                    last = (ci == _NCH - 1) & (w == _NWIN - 1)
                    do_flush = (jnp.max(c) > _PCAP - _CHECK) | last
                    pl.when(do_flush)(flush)
                    t = top[base + _K - 1]
                    c = jnp.where(do_flush, jnp.zeros_like(c), c)
                    return c, t

                c0 = cbuf[sb]
                t0 = top[base + _K - 1]
                c1, _t1 = lax.fori_loop(0, _NWIN, win_body, (c0, t0))
                cbuf[sb] = c1
                return carry

            lax.fori_loop(0, _SB, sb_body, 0)
            return carry

        lax.fori_loop(0, _NCH, chunk_body, 0)

        def pack_body(sb, carry):
            for i in range(_K):
                obuf[i, pl.ds(sb * _L, _L)] = top[sb * _K + i]
            return carry

        lax.fori_loop(0, _SB, pack_body, 0)
        pltpu.sync_copy(obuf, out_hbm.at[b, :, pl.ds(d0, _FB)])
        return carry

    lax.fori_loop(0, _GPW, group_body, 0)


_kmax = pl.kernel(
    _kmax_body,
    out_type=jax.ShapeDtypeStruct((_B, _K, _D), jnp.float32),
    mesh=plsc.VectorSubcoreMesh(
        core_axis_name="c", subcore_axis_name="s",
        num_cores=_NC, num_subcores=_NS,
    ),
    compiler_params=pltpu.CompilerParams(needs_layout_passes=False),
    scratch_types=[
        pltpu.VMEM((_CH, _FB), jnp.float32),
        pltpu.VMEM((_SB * _PCAP * _L,), jnp.float32),
        pltpu.VMEM((_SB * _K, _L), jnp.float32),
        pltpu.VMEM((_K, _FB), jnp.float32),
        pltpu.VMEM((_SB, _L), jnp.int32),
    ],
)


@jax.jit
def kernel(x):
    return _kmax(x)
